# baseline (speedup 1.0000x reference)
import functools

import jax
import jax.numpy as jnp
from jax import lax
from jax.experimental import pallas as pl
from jax.experimental.pallas import tpu as pltpu
from jax.experimental.pallas import tpu_sc as plsc

MIN_R = 1.0
MAX_R = 5.0

_NC = 2
_NS = 16
_NW = _NC * _NS
_CHUNK = 128
_PC = 1024


def _pack_body(a_ref, out_ref):
    r = lax.broadcasted_iota(jnp.int32, (64, 64), 0)
    c = lax.broadcasted_iota(jnp.int32, (64, 64), 1)
    eye = (r == c).astype(jnp.float32)
    dn = (((0,), (0,)), ((), ()))
    a = a_ref[...]
    out_ref[:, 0:64] = lax.dot_general(
        a[:, 0:512], eye, dn, preferred_element_type=jnp.float32)
    out_ref[:, 64:128] = lax.dot_general(
        a[:, 512:1024], eye, dn, preferred_element_type=jnp.float32)


def _tc_pack(tT):
    n = tT.shape[1]
    nblk = -(-n // _PC)
    return pl.pallas_call(
        _pack_body,
        grid=(nblk,),
        in_specs=[pl.BlockSpec((64, _PC), lambda i: (0, i))],
        out_specs=pl.BlockSpec((512, 128), lambda i: (i, 0)),
        out_shape=jax.ShapeDtypeStruct((nblk * 512, 128), jnp.float32),
    )(tT)


def _sc_gather(ut2, it2, u_idx3, i_idx3, B):
    n_chunks = u_idx3.shape[1]
    b_per_w = n_chunks * _CHUNK
    mesh = plsc.VectorSubcoreMesh(core_axis_name="c", subcore_axis_name="s")

    @functools.partial(
        pl.kernel,
        out_type=(
            jax.ShapeDtypeStruct((B, 128), jnp.float32),
            jax.ShapeDtypeStruct((B, 128), jnp.float32),
        ),
        mesh=mesh,
        compiler_params=pltpu.CompilerParams(use_tc_tiling_on_sc=True),
        scratch_types=[
            pltpu.VMEM((n_chunks, _CHUNK), jnp.int32),
            pltpu.VMEM((n_chunks, _CHUNK), jnp.int32),
            pltpu.VMEM((n_chunks * _CHUNK, 128), jnp.float32),
            pltpu.SemaphoreType.DMA,
        ],
    )
    def gather_kernel(ut_hbm, it_hbm, uidx_hbm, iidx_hbm, uout_hbm, iout_hbm,
                      uidx_v, iidx_v, rows_v, sem):
        wid = lax.axis_index("s") * _NC + lax.axis_index("c")
        base = wid * b_per_w
        pltpu.sync_copy(uidx_hbm.at[wid], uidx_v)
        pltpu.sync_copy(iidx_hbm.at[wid], iidx_v)

        def one_table(tbl_hbm, idx_v, out_hbm):
            copies = []
            for j in range(n_chunks):
                copies.append(pltpu.async_copy(
                    tbl_hbm.at[idx_v.at[j]],
                    rows_v.at[pl.ds(j * _CHUNK, _CHUNK)], sem))
            for c in copies:
                c.wait()
            pltpu.sync_copy(rows_v, out_hbm.at[pl.ds(base, b_per_w)])

        one_table(ut_hbm, uidx_v, uout_hbm)
        one_table(it_hbm, iidx_v, iout_hbm)

    return gather_kernel(ut2, it2, u_idx3, i_idx3)


def _mlp_body(ubuf_ref, ibuf_ref, upar_ref, ipar_ref, w0_ref, b0_ref, w1_ref,
              b1_ref, w2_ref, b2_ref, w3_ref, b3_ref, uout_ref, iout_ref,
              r_ref):
    ubuf = ubuf_ref[...]
    ibuf = ibuf_ref[...]
    u = jnp.where(upar_ref[...] == 0, ubuf[:, 0:64], ubuf[:, 64:128])
    i = jnp.where(ipar_ref[...] == 0, ibuf[:, 0:64], ibuf[:, 64:128])
    uout_ref[...] = u
    iout_ref[...] = i
    w0 = w0_ref[...]
    h = jnp.dot(u, w0[0:64], preferred_element_type=jnp.float32)
    h = h + jnp.dot(i, w0[64:128], preferred_element_type=jnp.float32)
    h = jax.nn.relu(h + b0_ref[...])
    h = jax.nn.relu(jnp.dot(h, w1_ref[...], preferred_element_type=jnp.float32)
                    + b1_ref[...])
    h = jax.nn.relu(jnp.dot(h, w2_ref[...], preferred_element_type=jnp.float32)
                    + b2_ref[...])
    r = jnp.sum(h * w3_ref[...], axis=1) + b3_ref[0, 0]
    r_ref[...] = jnp.clip(r, MIN_R, MAX_R)


def _tc_mlp(ubuf, ibuf, upar, ipar, W0, b0, W1, b1, W2, b2, W3, b3):
    B = ubuf.shape[0]
    D = 64
    TB = 2048
    grid = (B // TB,)
    b0r = b0.reshape(1, D)
    b1r = b1.reshape(1, D)
    b2r = b2.reshape(1, D)
    w3r = W3.reshape(1, D)
    b3r = b3.reshape(1, 1)
    return pl.pallas_call(
        _mlp_body,
        grid=grid,
        in_specs=[
            pl.BlockSpec((TB, 128), lambda i: (i, 0)),
            pl.BlockSpec((TB, 128), lambda i: (i, 0)),
            pl.BlockSpec((TB, 1), lambda i: (i, 0)),
            pl.BlockSpec((TB, 1), lambda i: (i, 0)),
            pl.BlockSpec((2 * D, D), lambda i: (0, 0)),
            pl.BlockSpec((1, D), lambda i: (0, 0)),
            pl.BlockSpec((D, D), lambda i: (0, 0)),
            pl.BlockSpec((1, D), lambda i: (0, 0)),
            pl.BlockSpec((D, D), lambda i: (0, 0)),
            pl.BlockSpec((1, D), lambda i: (0, 0)),
            pl.BlockSpec((1, D), lambda i: (0, 0)),
            pl.BlockSpec((1, 1), lambda i: (0, 0)),
        ],
        out_specs=[
            pl.BlockSpec((TB, D), lambda i: (i, 0)),
            pl.BlockSpec((TB, D), lambda i: (i, 0)),
            pl.BlockSpec((TB,), lambda i: (i,)),
        ],
        out_shape=[
            jax.ShapeDtypeStruct((B, D), jnp.float32),
            jax.ShapeDtypeStruct((B, D), jnp.float32),
            jax.ShapeDtypeStruct((B,), jnp.float32),
        ],
    )(ubuf, ibuf, upar, ipar, W0, b0r, W1, b1r, W2, b2r, w3r, b3r)


def kernel(U_ids, I_ids, user_table, item_table, W0, b0, W1, b1, W2, b2, W3, b3):
    B = U_ids.shape[0]
    ut2 = _tc_pack(user_table.T)
    it2 = _tc_pack(item_table.T)
    n_chunks = B // (_NW * _CHUNK)
    u_pk = (U_ids >> 10) * 512 + (U_ids & 511)
    i_pk = (I_ids >> 10) * 512 + (I_ids & 511)
    u_idx3 = u_pk.reshape(_NW, n_chunks, _CHUNK)
    i_idx3 = i_pk.reshape(_NW, n_chunks, _CHUNK)
    upar = ((U_ids >> 9) & 1).astype(jnp.int32).reshape(B, 1)
    ipar = ((I_ids >> 9) & 1).astype(jnp.int32).reshape(B, 1)
    ubuf, ibuf = _sc_gather(ut2, it2, u_idx3, i_idx3, B)
    U_emb, I_emb, R = _tc_mlp(ubuf, ibuf, upar, ipar,
                              W0, b0, W1, b1, W2, b2, W3, b3)
    return (U_emb, I_emb, R)


# speedup vs baseline: 2.3789x; 2.3789x over previous
import functools

import jax
import jax.numpy as jnp
from jax import lax
from jax.experimental import pallas as pl
from jax.experimental.pallas import tpu as pltpu
from jax.experimental.pallas import tpu_sc as plsc

MIN_R = 1.0
MAX_R = 5.0

_NC = 2
_NS = 16
_NW = _NC * _NS
_CHUNK = 128
_PC = 8192


def _pack_body(a_ref, out_ref):
    r = lax.broadcasted_iota(jnp.int32, (64, 64), 0)
    c = lax.broadcasted_iota(jnp.int32, (64, 64), 1)
    eye = (r == c).astype(jnp.float32)
    dn = (((0,), (0,)), ((), ()))
    a = a_ref[...]
    for k in range(_PC // 1024):
        sub = a[:, k * 1024:(k + 1) * 1024]
        out_ref[k * 512:(k + 1) * 512, 0:64] = lax.dot_general(
            sub[:, 0:512], eye, dn, preferred_element_type=jnp.float32)
        out_ref[k * 512:(k + 1) * 512, 64:128] = lax.dot_general(
            sub[:, 512:1024], eye, dn, preferred_element_type=jnp.float32)


def _tc_pack(tT):
    n = tT.shape[1]
    nblk = -(-n // _PC)
    return pl.pallas_call(
        _pack_body,
        grid=(nblk,),
        in_specs=[pl.BlockSpec((64, _PC), lambda i: (0, i))],
        out_specs=pl.BlockSpec((_PC // 2, 128), lambda i: (i, 0)),
        out_shape=jax.ShapeDtypeStruct((nblk * (_PC // 2), 128), jnp.float32),
    )(tT)


def _sc_gather(ut2, it2, u_idx3, i_idx3, B):
    n_chunks = u_idx3.shape[1]
    b_per_w = n_chunks * _CHUNK
    mesh = plsc.VectorSubcoreMesh(core_axis_name="c", subcore_axis_name="s")

    @functools.partial(
        pl.kernel,
        out_type=(
            jax.ShapeDtypeStruct((B, 128), jnp.float32),
            jax.ShapeDtypeStruct((B, 128), jnp.float32),
        ),
        mesh=mesh,
        compiler_params=pltpu.CompilerParams(use_tc_tiling_on_sc=True),
        scratch_types=[
            pltpu.VMEM((n_chunks, _CHUNK), jnp.int32),
            pltpu.VMEM((n_chunks, _CHUNK), jnp.int32),
            pltpu.VMEM((n_chunks * _CHUNK, 128), jnp.float32),
            pltpu.SemaphoreType.DMA,
        ],
    )
    def gather_kernel(ut_hbm, it_hbm, uidx_hbm, iidx_hbm, uout_hbm, iout_hbm,
                      uidx_v, iidx_v, rows_v, sem):
        wid = lax.axis_index("s") * _NC + lax.axis_index("c")
        base = wid * b_per_w
        pltpu.sync_copy(uidx_hbm.at[wid], uidx_v)
        pltpu.sync_copy(iidx_hbm.at[wid], iidx_v)

        def one_table(tbl_hbm, idx_v, out_hbm):
            copies = []
            for j in range(n_chunks):
                copies.append(pltpu.async_copy(
                    tbl_hbm.at[idx_v.at[j]],
                    rows_v.at[pl.ds(j * _CHUNK, _CHUNK)], sem))
            for c in copies:
                c.wait()
            pltpu.sync_copy(rows_v, out_hbm.at[pl.ds(base, b_per_w)])

        one_table(ut_hbm, uidx_v, uout_hbm)
        one_table(it_hbm, iidx_v, iout_hbm)

    return gather_kernel(ut2, it2, u_idx3, i_idx3)


def _mlp_body(ubuf_ref, ibuf_ref, upar_ref, ipar_ref, w0_ref, b0_ref, w1_ref,
              b1_ref, w2_ref, b2_ref, w3_ref, b3_ref, uout_ref, iout_ref,
              r_ref):
    ubuf = ubuf_ref[...]
    ibuf = ibuf_ref[...]
    u = jnp.where(upar_ref[...] == 0, ubuf[:, 0:64], ubuf[:, 64:128])
    i = jnp.where(ipar_ref[...] == 0, ibuf[:, 0:64], ibuf[:, 64:128])
    uout_ref[...] = u
    iout_ref[...] = i
    w0 = w0_ref[...]
    h = jnp.dot(u, w0[0:64], preferred_element_type=jnp.float32)
    h = h + jnp.dot(i, w0[64:128], preferred_element_type=jnp.float32)
    h = jax.nn.relu(h + b0_ref[...])
    h = jax.nn.relu(jnp.dot(h, w1_ref[...], preferred_element_type=jnp.float32)
                    + b1_ref[...])
    h = jax.nn.relu(jnp.dot(h, w2_ref[...], preferred_element_type=jnp.float32)
                    + b2_ref[...])
    r = jnp.sum(h * w3_ref[...], axis=1) + b3_ref[0, 0]
    r_ref[...] = jnp.clip(r, MIN_R, MAX_R)


def _tc_mlp(ubuf, ibuf, upar, ipar, W0, b0, W1, b1, W2, b2, W3, b3):
    B = ubuf.shape[0]
    D = 64
    TB = 2048
    grid = (B // TB,)
    b0r = b0.reshape(1, D)
    b1r = b1.reshape(1, D)
    b2r = b2.reshape(1, D)
    w3r = W3.reshape(1, D)
    b3r = b3.reshape(1, 1)
    return pl.pallas_call(
        _mlp_body,
        grid=grid,
        in_specs=[
            pl.BlockSpec((TB, 128), lambda i: (i, 0)),
            pl.BlockSpec((TB, 128), lambda i: (i, 0)),
            pl.BlockSpec((TB, 1), lambda i: (i, 0)),
            pl.BlockSpec((TB, 1), lambda i: (i, 0)),
            pl.BlockSpec((2 * D, D), lambda i: (0, 0)),
            pl.BlockSpec((1, D), lambda i: (0, 0)),
            pl.BlockSpec((D, D), lambda i: (0, 0)),
            pl.BlockSpec((1, D), lambda i: (0, 0)),
            pl.BlockSpec((D, D), lambda i: (0, 0)),
            pl.BlockSpec((1, D), lambda i: (0, 0)),
            pl.BlockSpec((1, D), lambda i: (0, 0)),
            pl.BlockSpec((1, 1), lambda i: (0, 0)),
        ],
        out_specs=[
            pl.BlockSpec((TB, D), lambda i: (i, 0)),
            pl.BlockSpec((TB, D), lambda i: (i, 0)),
            pl.BlockSpec((TB,), lambda i: (i,)),
        ],
        out_shape=[
            jax.ShapeDtypeStruct((B, D), jnp.float32),
            jax.ShapeDtypeStruct((B, D), jnp.float32),
            jax.ShapeDtypeStruct((B,), jnp.float32),
        ],
    )(ubuf, ibuf, upar, ipar, W0, b0r, W1, b1r, W2, b2r, w3r, b3r)


def kernel(U_ids, I_ids, user_table, item_table, W0, b0, W1, b1, W2, b2, W3, b3):
    B = U_ids.shape[0]
    ut2 = _tc_pack(user_table.T)
    it2 = _tc_pack(item_table.T)
    n_chunks = B // (_NW * _CHUNK)
    u_pk = (U_ids >> 10) * 512 + (U_ids & 511)
    i_pk = (I_ids >> 10) * 512 + (I_ids & 511)
    u_idx3 = u_pk.reshape(_NW, n_chunks, _CHUNK)
    i_idx3 = i_pk.reshape(_NW, n_chunks, _CHUNK)
    upar = ((U_ids >> 9) & 1).astype(jnp.int32).reshape(B, 1)
    ipar = ((I_ids >> 9) & 1).astype(jnp.int32).reshape(B, 1)
    ubuf, ibuf = _sc_gather(ut2, it2, u_idx3, i_idx3, B)
    U_emb, I_emb, R = _tc_mlp(ubuf, ibuf, upar, ipar,
                              W0, b0, W1, b1, W2, b2, W3, b3)
    return (U_emb, I_emb, R)


# speedup vs baseline: 2.6150x; 1.0993x over previous
import functools

import jax
import jax.numpy as jnp
from jax import lax
from jax.experimental import pallas as pl
from jax.experimental.pallas import tpu as pltpu
from jax.experimental.pallas import tpu_sc as plsc

MIN_R = 1.0
MAX_R = 5.0

_NC = 2
_NS = 16
_NW = _NC * _NS
_CHUNK = 128
_PC = 16384


def _pack_body(a_ref, out_ref):
    r = lax.broadcasted_iota(jnp.int32, (64, 64), 0)
    c = lax.broadcasted_iota(jnp.int32, (64, 64), 1)
    eye = (r == c).astype(jnp.float32)
    dn = (((0,), (0,)), ((), ()))
    a = a_ref[...]
    for k in range(_PC // 1024):
        sub = a[:, k * 1024:(k + 1) * 1024]
        out_ref[k * 512:(k + 1) * 512, 0:64] = lax.dot_general(
            sub[:, 0:512], eye, dn, preferred_element_type=jnp.float32)
        out_ref[k * 512:(k + 1) * 512, 64:128] = lax.dot_general(
            sub[:, 512:1024], eye, dn, preferred_element_type=jnp.float32)


def _tc_pack(tT):
    n = tT.shape[1]
    nblk = -(-n // _PC)
    return pl.pallas_call(
        _pack_body,
        grid=(nblk,),
        in_specs=[pl.BlockSpec((64, _PC), lambda i: (0, i))],
        out_specs=pl.BlockSpec((_PC // 2, 128), lambda i: (i, 0)),
        out_shape=jax.ShapeDtypeStruct((nblk * (_PC // 2), 128), jnp.float32),
    )(tT)


def _sc_gather(ut2, it2, u_idx3, i_idx3, B):
    n_chunks = u_idx3.shape[1]
    b_per_w = n_chunks * _CHUNK
    mesh = plsc.VectorSubcoreMesh(core_axis_name="c", subcore_axis_name="s")

    @functools.partial(
        pl.kernel,
        out_type=(
            jax.ShapeDtypeStruct((B, 128), jnp.float32),
            jax.ShapeDtypeStruct((B, 128), jnp.float32),
        ),
        mesh=mesh,
        compiler_params=pltpu.CompilerParams(use_tc_tiling_on_sc=True),
        scratch_types=[
            pltpu.VMEM((n_chunks, _CHUNK), jnp.int32),
            pltpu.VMEM((n_chunks, _CHUNK), jnp.int32),
            pltpu.VMEM((n_chunks * _CHUNK, 128), jnp.float32),
            pltpu.SemaphoreType.DMA,
        ],
    )
    def gather_kernel(ut_hbm, it_hbm, uidx_hbm, iidx_hbm, uout_hbm, iout_hbm,
                      uidx_v, iidx_v, rows_v, sem):
        wid = lax.axis_index("s") * _NC + lax.axis_index("c")
        base = wid * b_per_w
        pltpu.sync_copy(uidx_hbm.at[wid], uidx_v)
        pltpu.sync_copy(iidx_hbm.at[wid], iidx_v)

        def one_table(tbl_hbm, idx_v, out_hbm):
            copies = []
            for j in range(n_chunks):
                copies.append(pltpu.async_copy(
                    tbl_hbm.at[idx_v.at[j]],
                    rows_v.at[pl.ds(j * _CHUNK, _CHUNK)], sem))
            for c in copies:
                c.wait()
            pltpu.sync_copy(rows_v, out_hbm.at[pl.ds(base, b_per_w)])

        one_table(ut_hbm, uidx_v, uout_hbm)
        one_table(it_hbm, iidx_v, iout_hbm)

    return gather_kernel(ut2, it2, u_idx3, i_idx3)


def _mlp_body(ubuf_ref, ibuf_ref, upar_ref, ipar_ref, w0_ref, b0_ref, w1_ref,
              b1_ref, w2_ref, b2_ref, w3_ref, b3_ref, uout_ref, iout_ref,
              r_ref):
    ubuf = ubuf_ref[...]
    ibuf = ibuf_ref[...]
    u = jnp.where(upar_ref[...] == 0, ubuf[:, 0:64], ubuf[:, 64:128])
    i = jnp.where(ipar_ref[...] == 0, ibuf[:, 0:64], ibuf[:, 64:128])
    uout_ref[...] = u
    iout_ref[...] = i
    w0 = w0_ref[...]
    h = jnp.dot(u, w0[0:64], preferred_element_type=jnp.float32)
    h = h + jnp.dot(i, w0[64:128], preferred_element_type=jnp.float32)
    h = jax.nn.relu(h + b0_ref[...])
    h = jax.nn.relu(jnp.dot(h, w1_ref[...], preferred_element_type=jnp.float32)
                    + b1_ref[...])
    h = jax.nn.relu(jnp.dot(h, w2_ref[...], preferred_element_type=jnp.float32)
                    + b2_ref[...])
    r = jnp.sum(h * w3_ref[...], axis=1) + b3_ref[0, 0]
    r_ref[...] = jnp.clip(r, MIN_R, MAX_R)


def _tc_mlp(ubuf, ibuf, upar, ipar, W0, b0, W1, b1, W2, b2, W3, b3):
    B = ubuf.shape[0]
    D = 64
    TB = 2048
    grid = (B // TB,)
    b0r = b0.reshape(1, D)
    b1r = b1.reshape(1, D)
    b2r = b2.reshape(1, D)
    w3r = W3.reshape(1, D)
    b3r = b3.reshape(1, 1)
    return pl.pallas_call(
        _mlp_body,
        grid=grid,
        in_specs=[
            pl.BlockSpec((TB, 128), lambda i: (i, 0)),
            pl.BlockSpec((TB, 128), lambda i: (i, 0)),
            pl.BlockSpec((TB, 1), lambda i: (i, 0)),
            pl.BlockSpec((TB, 1), lambda i: (i, 0)),
            pl.BlockSpec((2 * D, D), lambda i: (0, 0)),
            pl.BlockSpec((1, D), lambda i: (0, 0)),
            pl.BlockSpec((D, D), lambda i: (0, 0)),
            pl.BlockSpec((1, D), lambda i: (0, 0)),
            pl.BlockSpec((D, D), lambda i: (0, 0)),
            pl.BlockSpec((1, D), lambda i: (0, 0)),
            pl.BlockSpec((1, D), lambda i: (0, 0)),
            pl.BlockSpec((1, 1), lambda i: (0, 0)),
        ],
        out_specs=[
            pl.BlockSpec((TB, D), lambda i: (i, 0)),
            pl.BlockSpec((TB, D), lambda i: (i, 0)),
            pl.BlockSpec((TB,), lambda i: (i,)),
        ],
        out_shape=[
            jax.ShapeDtypeStruct((B, D), jnp.float32),
            jax.ShapeDtypeStruct((B, D), jnp.float32),
            jax.ShapeDtypeStruct((B,), jnp.float32),
        ],
    )(ubuf, ibuf, upar, ipar, W0, b0r, W1, b1r, W2, b2r, w3r, b3r)


def kernel(U_ids, I_ids, user_table, item_table, W0, b0, W1, b1, W2, b2, W3, b3):
    B = U_ids.shape[0]
    ut2 = _tc_pack(user_table.T)
    it2 = _tc_pack(item_table.T)
    n_chunks = B // (_NW * _CHUNK)
    u_pk = (U_ids >> 10) * 512 + (U_ids & 511)
    i_pk = (I_ids >> 10) * 512 + (I_ids & 511)
    u_idx3 = u_pk.reshape(_NW, n_chunks, _CHUNK)
    i_idx3 = i_pk.reshape(_NW, n_chunks, _CHUNK)
    upar = ((U_ids >> 9) & 1).astype(jnp.int32).reshape(B, 1)
    ipar = ((I_ids >> 9) & 1).astype(jnp.int32).reshape(B, 1)
    ubuf, ibuf = _sc_gather(ut2, it2, u_idx3, i_idx3, B)
    U_emb, I_emb, R = _tc_mlp(ubuf, ibuf, upar, ipar,
                              W0, b0, W1, b1, W2, b2, W3, b3)
    return (U_emb, I_emb, R)


# speedup vs baseline: 2.6907x; 1.0289x over previous
import functools

import jax
import jax.numpy as jnp
from jax import lax
from jax.experimental import pallas as pl
from jax.experimental.pallas import tpu as pltpu
from jax.experimental.pallas import tpu_sc as plsc

MIN_R = 1.0
MAX_R = 5.0

_NC = 2
_NS = 16
_NW = _NC * _NS
_CHUNK = 128
_PC = 32768


def _pack_body(a_ref, out_ref):
    r = lax.broadcasted_iota(jnp.int32, (64, 64), 0)
    c = lax.broadcasted_iota(jnp.int32, (64, 64), 1)
    eye = (r == c).astype(jnp.float32)
    dn = (((0,), (0,)), ((), ()))
    a = a_ref[...]
    for k in range(_PC // 1024):
        sub = a[:, k * 1024:(k + 1) * 1024]
        out_ref[k * 512:(k + 1) * 512, 0:64] = lax.dot_general(
            sub[:, 0:512], eye, dn, preferred_element_type=jnp.float32)
        out_ref[k * 512:(k + 1) * 512, 64:128] = lax.dot_general(
            sub[:, 512:1024], eye, dn, preferred_element_type=jnp.float32)


def _tc_pack(tT):
    n = tT.shape[1]
    nblk = -(-n // _PC)
    return pl.pallas_call(
        _pack_body,
        grid=(nblk,),
        in_specs=[pl.BlockSpec((64, _PC), lambda i: (0, i))],
        out_specs=pl.BlockSpec((_PC // 2, 128), lambda i: (i, 0)),
        out_shape=jax.ShapeDtypeStruct((nblk * (_PC // 2), 128), jnp.float32),
    )(tT)


def _sc_gather(ut2, it2, u_idx3, i_idx3, B):
    n_chunks = u_idx3.shape[1]
    b_per_w = n_chunks * _CHUNK
    mesh = plsc.VectorSubcoreMesh(core_axis_name="c", subcore_axis_name="s")

    @functools.partial(
        pl.kernel,
        out_type=(
            jax.ShapeDtypeStruct((B, 128), jnp.float32),
            jax.ShapeDtypeStruct((B, 128), jnp.float32),
        ),
        mesh=mesh,
        compiler_params=pltpu.CompilerParams(use_tc_tiling_on_sc=True),
        scratch_types=[
            pltpu.VMEM((n_chunks, _CHUNK), jnp.int32),
            pltpu.VMEM((n_chunks, _CHUNK), jnp.int32),
            pltpu.VMEM((n_chunks * _CHUNK, 128), jnp.float32),
            pltpu.SemaphoreType.DMA,
        ],
    )
    def gather_kernel(ut_hbm, it_hbm, uidx_hbm, iidx_hbm, uout_hbm, iout_hbm,
                      uidx_v, iidx_v, rows_v, sem):
        wid = lax.axis_index("s") * _NC + lax.axis_index("c")
        base = wid * b_per_w
        pltpu.sync_copy(uidx_hbm.at[wid], uidx_v)
        pltpu.sync_copy(iidx_hbm.at[wid], iidx_v)

        def one_table(tbl_hbm, idx_v, out_hbm):
            copies = []
            for j in range(n_chunks):
                copies.append(pltpu.async_copy(
                    tbl_hbm.at[idx_v.at[j]],
                    rows_v.at[pl.ds(j * _CHUNK, _CHUNK)], sem))
            for c in copies:
                c.wait()
            pltpu.sync_copy(rows_v, out_hbm.at[pl.ds(base, b_per_w)])

        one_table(ut_hbm, uidx_v, uout_hbm)
        one_table(it_hbm, iidx_v, iout_hbm)

    return gather_kernel(ut2, it2, u_idx3, i_idx3)


def _mlp_body(ubuf_ref, ibuf_ref, upar_ref, ipar_ref, w0_ref, b0_ref, w1_ref,
              b1_ref, w2_ref, b2_ref, w3_ref, b3_ref, uout_ref, iout_ref,
              r_ref):
    ubuf = ubuf_ref[...]
    ibuf = ibuf_ref[...]
    u = jnp.where(upar_ref[...] == 0, ubuf[:, 0:64], ubuf[:, 64:128])
    i = jnp.where(ipar_ref[...] == 0, ibuf[:, 0:64], ibuf[:, 64:128])
    uout_ref[...] = u
    iout_ref[...] = i
    w0 = w0_ref[...]
    h = jnp.dot(u, w0[0:64], preferred_element_type=jnp.float32)
    h = h + jnp.dot(i, w0[64:128], preferred_element_type=jnp.float32)
    h = jax.nn.relu(h + b0_ref[...])
    h = jax.nn.relu(jnp.dot(h, w1_ref[...], preferred_element_type=jnp.float32)
                    + b1_ref[...])
    h = jax.nn.relu(jnp.dot(h, w2_ref[...], preferred_element_type=jnp.float32)
                    + b2_ref[...])
    r = jnp.sum(h * w3_ref[...], axis=1) + b3_ref[0, 0]
    r_ref[...] = jnp.clip(r, MIN_R, MAX_R)


def _tc_mlp(ubuf, ibuf, upar, ipar, W0, b0, W1, b1, W2, b2, W3, b3):
    B = ubuf.shape[0]
    D = 64
    TB = 2048
    grid = (B // TB,)
    b0r = b0.reshape(1, D)
    b1r = b1.reshape(1, D)
    b2r = b2.reshape(1, D)
    w3r = W3.reshape(1, D)
    b3r = b3.reshape(1, 1)
    return pl.pallas_call(
        _mlp_body,
        grid=grid,
        in_specs=[
            pl.BlockSpec((TB, 128), lambda i: (i, 0)),
            pl.BlockSpec((TB, 128), lambda i: (i, 0)),
            pl.BlockSpec((TB, 1), lambda i: (i, 0)),
            pl.BlockSpec((TB, 1), lambda i: (i, 0)),
            pl.BlockSpec((2 * D, D), lambda i: (0, 0)),
            pl.BlockSpec((1, D), lambda i: (0, 0)),
            pl.BlockSpec((D, D), lambda i: (0, 0)),
            pl.BlockSpec((1, D), lambda i: (0, 0)),
            pl.BlockSpec((D, D), lambda i: (0, 0)),
            pl.BlockSpec((1, D), lambda i: (0, 0)),
            pl.BlockSpec((1, D), lambda i: (0, 0)),
            pl.BlockSpec((1, 1), lambda i: (0, 0)),
        ],
        out_specs=[
            pl.BlockSpec((TB, D), lambda i: (i, 0)),
            pl.BlockSpec((TB, D), lambda i: (i, 0)),
            pl.BlockSpec((TB,), lambda i: (i,)),
        ],
        out_shape=[
            jax.ShapeDtypeStruct((B, D), jnp.float32),
            jax.ShapeDtypeStruct((B, D), jnp.float32),
            jax.ShapeDtypeStruct((B,), jnp.float32),
        ],
    )(ubuf, ibuf, upar, ipar, W0, b0r, W1, b1r, W2, b2r, w3r, b3r)


def kernel(U_ids, I_ids, user_table, item_table, W0, b0, W1, b1, W2, b2, W3, b3):
    B = U_ids.shape[0]
    ut2 = _tc_pack(user_table.T)
    it2 = _tc_pack(item_table.T)
    n_chunks = B // (_NW * _CHUNK)
    u_pk = (U_ids >> 10) * 512 + (U_ids & 511)
    i_pk = (I_ids >> 10) * 512 + (I_ids & 511)
    u_idx3 = u_pk.reshape(_NW, n_chunks, _CHUNK)
    i_idx3 = i_pk.reshape(_NW, n_chunks, _CHUNK)
    upar = ((U_ids >> 9) & 1).astype(jnp.int32).reshape(B, 1)
    ipar = ((I_ids >> 9) & 1).astype(jnp.int32).reshape(B, 1)
    ubuf, ibuf = _sc_gather(ut2, it2, u_idx3, i_idx3, B)
    U_emb, I_emb, R = _tc_mlp(ubuf, ibuf, upar, ipar,
                              W0, b0, W1, b1, W2, b2, W3, b3)
    return (U_emb, I_emb, R)


# speedup vs baseline: 2.7613x; 1.0262x over previous
import functools

import jax
import jax.numpy as jnp
from jax import lax
from jax.experimental import pallas as pl
from jax.experimental.pallas import tpu as pltpu
from jax.experimental.pallas import tpu_sc as plsc

MIN_R = 1.0
MAX_R = 5.0

_NC = 2
_NS = 16
_NW = _NC * _NS
_CHUNK = 128
_PC = 32768


def _pack_body(a_ref, out_ref):
    a = a_ref[...]
    for k in range(_PC // 1024):
        sub = a[:, k * 1024:(k + 1) * 1024]
        out_ref[k * 512:(k + 1) * 512, 0:64] = sub[:, 0:512].T
        out_ref[k * 512:(k + 1) * 512, 64:128] = sub[:, 512:1024].T


def _tc_pack(tT):
    n = tT.shape[1]
    nblk = -(-n // _PC)
    return pl.pallas_call(
        _pack_body,
        grid=(nblk,),
        in_specs=[pl.BlockSpec((64, _PC), lambda i: (0, i))],
        out_specs=pl.BlockSpec((_PC // 2, 128), lambda i: (i, 0)),
        out_shape=jax.ShapeDtypeStruct((nblk * (_PC // 2), 128), jnp.float32),
    )(tT)


def _sc_gather(t2, idx3, B):
    n_chunks = idx3.shape[1]
    b_per_w = n_chunks * _CHUNK
    mesh = plsc.VectorSubcoreMesh(core_axis_name="c", subcore_axis_name="s")

    @functools.partial(
        pl.kernel,
        out_type=jax.ShapeDtypeStruct((B, 128), jnp.float32),
        mesh=mesh,
        compiler_params=pltpu.CompilerParams(use_tc_tiling_on_sc=True),
        scratch_types=[
            pltpu.VMEM((n_chunks, _CHUNK), jnp.int32),
            pltpu.VMEM((n_chunks * _CHUNK, 128), jnp.float32),
            pltpu.SemaphoreType.DMA,
        ],
    )
    def gather_kernel(tbl_hbm, idx_hbm, out_hbm, idx_v, rows_v, sem):
        wid = lax.axis_index("s") * _NC + lax.axis_index("c")
        base = wid * b_per_w
        pltpu.sync_copy(idx_hbm.at[wid], idx_v)
        copies = []
        for j in range(n_chunks):
            copies.append(pltpu.async_copy(
                tbl_hbm.at[idx_v.at[j]],
                rows_v.at[pl.ds(j * _CHUNK, _CHUNK)], sem))
        for c in copies:
            c.wait()
        pltpu.sync_copy(rows_v, out_hbm.at[pl.ds(base, b_per_w)])

    return gather_kernel(t2, idx3)


def _mlp_body(ubuf_ref, ibuf_ref, upar_ref, ipar_ref, w0_ref, b0_ref, w1_ref,
              b1_ref, w2_ref, b2_ref, w3_ref, b3_ref, uout_ref, iout_ref,
              r_ref):
    ubuf = ubuf_ref[...]
    ibuf = ibuf_ref[...]
    u = jnp.where(upar_ref[...] == 0, ubuf[:, 0:64], ubuf[:, 64:128])
    i = jnp.where(ipar_ref[...] == 0, ibuf[:, 0:64], ibuf[:, 64:128])
    uout_ref[...] = u
    iout_ref[...] = i
    w0 = w0_ref[...]
    h = jnp.dot(u, w0[0:64], preferred_element_type=jnp.float32)
    h = h + jnp.dot(i, w0[64:128], preferred_element_type=jnp.float32)
    h = jax.nn.relu(h + b0_ref[...])
    h = jax.nn.relu(jnp.dot(h, w1_ref[...], preferred_element_type=jnp.float32)
                    + b1_ref[...])
    h = jax.nn.relu(jnp.dot(h, w2_ref[...], preferred_element_type=jnp.float32)
                    + b2_ref[...])
    r = jnp.sum(h * w3_ref[...], axis=1) + b3_ref[0, 0]
    r_ref[...] = jnp.clip(r, MIN_R, MAX_R)


def _tc_mlp(ubuf, ibuf, upar, ipar, W0, b0, W1, b1, W2, b2, W3, b3):
    B = ubuf.shape[0]
    D = 64
    TB = 2048
    grid = (B // TB,)
    b0r = b0.reshape(1, D)
    b1r = b1.reshape(1, D)
    b2r = b2.reshape(1, D)
    w3r = W3.reshape(1, D)
    b3r = b3.reshape(1, 1)
    return pl.pallas_call(
        _mlp_body,
        grid=grid,
        in_specs=[
            pl.BlockSpec((TB, 128), lambda i: (i, 0)),
            pl.BlockSpec((TB, 128), lambda i: (i, 0)),
            pl.BlockSpec((TB, 1), lambda i: (i, 0)),
            pl.BlockSpec((TB, 1), lambda i: (i, 0)),
            pl.BlockSpec((2 * D, D), lambda i: (0, 0)),
            pl.BlockSpec((1, D), lambda i: (0, 0)),
            pl.BlockSpec((D, D), lambda i: (0, 0)),
            pl.BlockSpec((1, D), lambda i: (0, 0)),
            pl.BlockSpec((D, D), lambda i: (0, 0)),
            pl.BlockSpec((1, D), lambda i: (0, 0)),
            pl.BlockSpec((1, D), lambda i: (0, 0)),
            pl.BlockSpec((1, 1), lambda i: (0, 0)),
        ],
        out_specs=[
            pl.BlockSpec((TB, D), lambda i: (i, 0)),
            pl.BlockSpec((TB, D), lambda i: (i, 0)),
            pl.BlockSpec((TB,), lambda i: (i,)),
        ],
        out_shape=[
            jax.ShapeDtypeStruct((B, D), jnp.float32),
            jax.ShapeDtypeStruct((B, D), jnp.float32),
            jax.ShapeDtypeStruct((B,), jnp.float32),
        ],
    )(ubuf, ibuf, upar, ipar, W0, b0r, W1, b1r, W2, b2r, w3r, b3r)


def kernel(U_ids, I_ids, user_table, item_table, W0, b0, W1, b1, W2, b2, W3, b3):
    B = U_ids.shape[0]
    ut2 = _tc_pack(user_table.T)
    it2 = _tc_pack(item_table.T)
    n_chunks = B // (_NW * _CHUNK)
    u_pk = (U_ids >> 10) * 512 + (U_ids & 511)
    i_pk = (I_ids >> 10) * 512 + (I_ids & 511)
    u_idx3 = u_pk.reshape(_NW, n_chunks, _CHUNK)
    i_idx3 = i_pk.reshape(_NW, n_chunks, _CHUNK)
    upar = ((U_ids >> 9) & 1).astype(jnp.int32).reshape(B, 1)
    ipar = ((I_ids >> 9) & 1).astype(jnp.int32).reshape(B, 1)
    ubuf = _sc_gather(ut2, u_idx3, B)
    ibuf = _sc_gather(it2, i_idx3, B)
    U_emb, I_emb, R = _tc_mlp(ubuf, ibuf, upar, ipar,
                              W0, b0, W1, b1, W2, b2, W3, b3)
    return (U_emb, I_emb, R)


# speedup vs baseline: 2.7703x; 1.0033x over previous
import functools

import jax
import jax.numpy as jnp
from jax import lax
from jax.experimental import pallas as pl
from jax.experimental.pallas import tpu as pltpu
from jax.experimental.pallas import tpu_sc as plsc

MIN_R = 1.0
MAX_R = 5.0

_NC = 2
_NS = 16
_NW = _NC * _NS
_CHUNK = 128
_PC = 32768


def _pack_body(a_ref, out_ref):
    a = a_ref[...]
    for k in range(_PC // 1024):
        sub = a[:, k * 1024:(k + 1) * 1024]
        out_ref[k * 512:(k + 1) * 512, 0:64] = sub[:, 0:512].T
        out_ref[k * 512:(k + 1) * 512, 64:128] = sub[:, 512:1024].T


def _tc_pack(tT):
    n = tT.shape[1]
    nblk = -(-n // _PC)
    return pl.pallas_call(
        _pack_body,
        grid=(nblk,),
        in_specs=[pl.BlockSpec((64, _PC), lambda i: (0, i))],
        out_specs=pl.BlockSpec((_PC // 2, 128), lambda i: (i, 0)),
        out_shape=jax.ShapeDtypeStruct((nblk * (_PC // 2), 128), jnp.float32),
    )(tT)


def _sc_gather(t2, idx3, B):
    n_chunks = idx3.shape[1]
    b_per_w = n_chunks * _CHUNK
    mesh = plsc.VectorSubcoreMesh(core_axis_name="c", subcore_axis_name="s")

    @functools.partial(
        pl.kernel,
        out_type=jax.ShapeDtypeStruct((B, 128), jnp.float32),
        mesh=mesh,
        compiler_params=pltpu.CompilerParams(use_tc_tiling_on_sc=True),
        scratch_types=[
            pltpu.VMEM((n_chunks, _CHUNK), jnp.int32),
            pltpu.VMEM((n_chunks * _CHUNK, 128), jnp.float32),
            pltpu.SemaphoreType.DMA,
        ],
    )
    def gather_kernel(tbl_hbm, idx_hbm, out_hbm, idx_v, rows_v, sem):
        wid = lax.axis_index("s") * _NC + lax.axis_index("c")
        base = wid * b_per_w
        pltpu.sync_copy(idx_hbm.at[wid], idx_v)
        copies = []
        for j in range(n_chunks):
            copies.append(pltpu.async_copy(
                tbl_hbm.at[idx_v.at[j]],
                rows_v.at[pl.ds(j * _CHUNK, _CHUNK)], sem))
        for c in copies:
            c.wait()
        pltpu.sync_copy(rows_v, out_hbm.at[pl.ds(base, b_per_w)])

    return gather_kernel(t2, idx3)


def _mlp_body(ubuf_ref, ibuf_ref, upar_ref, ipar_ref, w0_ref, b0_ref, w1_ref,
              b1_ref, w2_ref, b2_ref, w3_ref, b3_ref, uout_ref, iout_ref,
              r_ref):
    ubuf = ubuf_ref[...]
    ibuf = ibuf_ref[...]
    u = jnp.where(upar_ref[...] == 0, ubuf[:, 0:64], ubuf[:, 64:128])
    i = jnp.where(ipar_ref[...] == 0, ibuf[:, 0:64], ibuf[:, 64:128])
    uout_ref[...] = u
    iout_ref[...] = i
    w0 = w0_ref[...]
    h = jnp.dot(u, w0[0:64], preferred_element_type=jnp.float32)
    h = h + jnp.dot(i, w0[64:128], preferred_element_type=jnp.float32)
    h = jax.nn.relu(h + b0_ref[...])
    h = jax.nn.relu(jnp.dot(h, w1_ref[...], preferred_element_type=jnp.float32)
                    + b1_ref[...])
    h = jax.nn.relu(jnp.dot(h, w2_ref[...], preferred_element_type=jnp.float32)
                    + b2_ref[...])
    r = jnp.sum(h * w3_ref[...], axis=1) + b3_ref[0, 0]
    r_ref[...] = jnp.clip(r, MIN_R, MAX_R)


def _tc_mlp(ubuf, ibuf, upar, ipar, W0, b0, W1, b1, W2, b2, W3, b3):
    B = ubuf.shape[0]
    D = 64
    TB = 4096
    grid = (B // TB,)
    b0r = b0.reshape(1, D)
    b1r = b1.reshape(1, D)
    b2r = b2.reshape(1, D)
    w3r = W3.reshape(1, D)
    b3r = b3.reshape(1, 1)
    return pl.pallas_call(
        _mlp_body,
        grid=grid,
        in_specs=[
            pl.BlockSpec((TB, 128), lambda i: (i, 0)),
            pl.BlockSpec((TB, 128), lambda i: (i, 0)),
            pl.BlockSpec((TB, 1), lambda i: (i, 0)),
            pl.BlockSpec((TB, 1), lambda i: (i, 0)),
            pl.BlockSpec((2 * D, D), lambda i: (0, 0)),
            pl.BlockSpec((1, D), lambda i: (0, 0)),
            pl.BlockSpec((D, D), lambda i: (0, 0)),
            pl.BlockSpec((1, D), lambda i: (0, 0)),
            pl.BlockSpec((D, D), lambda i: (0, 0)),
            pl.BlockSpec((1, D), lambda i: (0, 0)),
            pl.BlockSpec((1, D), lambda i: (0, 0)),
            pl.BlockSpec((1, 1), lambda i: (0, 0)),
        ],
        out_specs=[
            pl.BlockSpec((TB, D), lambda i: (i, 0)),
            pl.BlockSpec((TB, D), lambda i: (i, 0)),
            pl.BlockSpec((TB,), lambda i: (i,)),
        ],
        out_shape=[
            jax.ShapeDtypeStruct((B, D), jnp.float32),
            jax.ShapeDtypeStruct((B, D), jnp.float32),
            jax.ShapeDtypeStruct((B,), jnp.float32),
        ],
    )(ubuf, ibuf, upar, ipar, W0, b0r, W1, b1r, W2, b2r, w3r, b3r)


def kernel(U_ids, I_ids, user_table, item_table, W0, b0, W1, b1, W2, b2, W3, b3):
    B = U_ids.shape[0]
    ut2 = _tc_pack(user_table.T)
    it2 = _tc_pack(item_table.T)
    n_chunks = B // (_NW * _CHUNK)
    u_pk = (U_ids >> 10) * 512 + (U_ids & 511)
    i_pk = (I_ids >> 10) * 512 + (I_ids & 511)
    u_idx3 = u_pk.reshape(_NW, n_chunks, _CHUNK)
    i_idx3 = i_pk.reshape(_NW, n_chunks, _CHUNK)
    upar = ((U_ids >> 9) & 1).astype(jnp.int32).reshape(B, 1)
    ipar = ((I_ids >> 9) & 1).astype(jnp.int32).reshape(B, 1)
    ubuf = _sc_gather(ut2, u_idx3, B)
    ibuf = _sc_gather(it2, i_idx3, B)
    U_emb, I_emb, R = _tc_mlp(ubuf, ibuf, upar, ipar,
                              W0, b0, W1, b1, W2, b2, W3, b3)
    return (U_emb, I_emb, R)


# speedup vs baseline: 2.8903x; 1.0433x over previous
import functools

import jax
import jax.numpy as jnp
from jax import lax
from jax.experimental import pallas as pl
from jax.experimental.pallas import tpu as pltpu
from jax.experimental.pallas import tpu_sc as plsc

MIN_R = 1.0
MAX_R = 5.0

_NC = 2
_NS = 16
_NW = _NC * _NS
_CHUNK = 128
_PC = 32768


def _pack_body(a_ref, out_ref):
    a = a_ref[...]
    for k in range(_PC // 1024):
        sub = a[:, k * 1024:(k + 1) * 1024]
        out_ref[k * 512:(k + 1) * 512, 0:64] = sub[:, 0:512].T
        out_ref[k * 512:(k + 1) * 512, 64:128] = sub[:, 512:1024].T


def _tc_pack(tT):
    n = tT.shape[1]
    nblk = -(-n // _PC)
    return pl.pallas_call(
        _pack_body,
        grid=(nblk,),
        in_specs=[pl.BlockSpec((64, _PC), lambda i: (0, i))],
        out_specs=pl.BlockSpec((_PC // 2, 128), lambda i: (i, 0)),
        out_shape=jax.ShapeDtypeStruct((nblk * (_PC // 2), 128), jnp.float32),
    )(tT)


def _sc_gather(t2, idx3, B):
    n_chunks = idx3.shape[1]
    b_per_w = n_chunks * _CHUNK
    mesh = plsc.VectorSubcoreMesh(core_axis_name="c", subcore_axis_name="s")

    @functools.partial(
        pl.kernel,
        out_type=jax.ShapeDtypeStruct((B, 128), jnp.float32),
        mesh=mesh,
        compiler_params=pltpu.CompilerParams(use_tc_tiling_on_sc=True),
        scratch_types=[
            pltpu.VMEM((n_chunks, _CHUNK), jnp.int32),
            pltpu.VMEM((n_chunks * _CHUNK, 128), jnp.float32),
            pltpu.SemaphoreType.DMA,
        ],
    )
    def gather_kernel(tbl_hbm, idx_hbm, out_hbm, idx_v, rows_v, sem):
        wid = lax.axis_index("s") * _NC + lax.axis_index("c")
        base = wid * b_per_w
        pltpu.sync_copy(idx_hbm.at[wid], idx_v)
        copies = []
        for j in range(n_chunks):
            copies.append(pltpu.async_copy(
                tbl_hbm.at[idx_v.at[j]],
                rows_v.at[pl.ds(j * _CHUNK, _CHUNK)], sem))
        for c in copies:
            c.wait()
        pltpu.sync_copy(rows_v, out_hbm.at[pl.ds(base, b_per_w)])

    return gather_kernel(t2, idx3)


def _mlp_body(ubuf_ref, ibuf_ref, upar_ref, ipar_ref, w0_ref, b0_ref, w1_ref,
              b1_ref, w2_ref, b2_ref, w3_ref, b3_ref, uout_ref, iout_ref,
              r_ref):
    ubuf = ubuf_ref[...]
    ibuf = ibuf_ref[...]
    u = jnp.where(upar_ref[...] == 0, ubuf[:, 0:64], ubuf[:, 64:128])
    i = jnp.where(ipar_ref[...] == 0, ibuf[:, 0:64], ibuf[:, 64:128])
    uout_ref[...] = u.T
    iout_ref[...] = i.T
    w0 = w0_ref[...]
    h = jnp.dot(u, w0[0:64], preferred_element_type=jnp.float32)
    h = h + jnp.dot(i, w0[64:128], preferred_element_type=jnp.float32)
    h = jax.nn.relu(h + b0_ref[...])
    h = jax.nn.relu(jnp.dot(h, w1_ref[...], preferred_element_type=jnp.float32)
                    + b1_ref[...])
    h = jax.nn.relu(jnp.dot(h, w2_ref[...], preferred_element_type=jnp.float32)
                    + b2_ref[...])
    r = jnp.sum(h * w3_ref[...], axis=1) + b3_ref[0, 0]
    r_ref[...] = jnp.clip(r, MIN_R, MAX_R)


def _tc_mlp(ubuf, ibuf, upar, ipar, W0, b0, W1, b1, W2, b2, W3, b3):
    B = ubuf.shape[0]
    D = 64
    TB = 4096
    grid = (B // TB,)
    b0r = b0.reshape(1, D)
    b1r = b1.reshape(1, D)
    b2r = b2.reshape(1, D)
    w3r = W3.reshape(1, D)
    b3r = b3.reshape(1, 1)
    return pl.pallas_call(
        _mlp_body,
        grid=grid,
        in_specs=[
            pl.BlockSpec((TB, 128), lambda i: (i, 0)),
            pl.BlockSpec((TB, 128), lambda i: (i, 0)),
            pl.BlockSpec((TB, 1), lambda i: (i, 0)),
            pl.BlockSpec((TB, 1), lambda i: (i, 0)),
            pl.BlockSpec((2 * D, D), lambda i: (0, 0)),
            pl.BlockSpec((1, D), lambda i: (0, 0)),
            pl.BlockSpec((D, D), lambda i: (0, 0)),
            pl.BlockSpec((1, D), lambda i: (0, 0)),
            pl.BlockSpec((D, D), lambda i: (0, 0)),
            pl.BlockSpec((1, D), lambda i: (0, 0)),
            pl.BlockSpec((1, D), lambda i: (0, 0)),
            pl.BlockSpec((1, 1), lambda i: (0, 0)),
        ],
        out_specs=[
            pl.BlockSpec((D, TB), lambda i: (0, i)),
            pl.BlockSpec((D, TB), lambda i: (0, i)),
            pl.BlockSpec((TB,), lambda i: (i,)),
        ],
        out_shape=[
            jax.ShapeDtypeStruct((D, B), jnp.float32),
            jax.ShapeDtypeStruct((D, B), jnp.float32),
            jax.ShapeDtypeStruct((B,), jnp.float32),
        ],
    )(ubuf, ibuf, upar, ipar, W0, b0r, W1, b1r, W2, b2r, w3r, b3r)


def kernel(U_ids, I_ids, user_table, item_table, W0, b0, W1, b1, W2, b2, W3, b3):
    B = U_ids.shape[0]
    ut2 = _tc_pack(user_table.T)
    it2 = _tc_pack(item_table.T)
    n_chunks = B // (_NW * _CHUNK)
    u_pk = (U_ids >> 10) * 512 + (U_ids & 511)
    i_pk = (I_ids >> 10) * 512 + (I_ids & 511)
    u_idx3 = u_pk.reshape(_NW, n_chunks, _CHUNK)
    i_idx3 = i_pk.reshape(_NW, n_chunks, _CHUNK)
    upar = ((U_ids >> 9) & 1).astype(jnp.int32).reshape(B, 1)
    ipar = ((I_ids >> 9) & 1).astype(jnp.int32).reshape(B, 1)
    ubuf = _sc_gather(ut2, u_idx3, B)
    ibuf = _sc_gather(it2, i_idx3, B)
    U_embT, I_embT, R = _tc_mlp(ubuf, ibuf, upar, ipar,
                                W0, b0, W1, b1, W2, b2, W3, b3)
    return (U_embT.T, I_embT.T, R)
